# Initial kernel scaffold; baseline (speedup 1.0000x reference)
import jax
import jax.numpy as jnp
from jax.experimental import pallas as pl


def kernel(x, edge_index, edge_type, W_rel1, W_root1, b1, W_rel2, W_root2, b2):
    raise NotImplementedError("write your pallas kernel here")


# speedup vs baseline: 11.5647x; 11.5647x over previous
import functools

import jax
import jax.numpy as jnp
from jax import lax
from jax.experimental import pallas as pl
from jax.experimental.pallas import tpu as pltpu
from jax.experimental.pallas import tpu_sc as plsc

N = 10000
E = 320000
D = 128
DH = 64
NPAD = 10240
NC, NS = 2, 16
NT = NC * NS
K = 80
CPT = E // (NT * K)
RPT = NPAD // NS
NJ = RPT // K
BN = 1024


def _sc_agg_body(hflat, srch, dsth, eth, zrh, z8h, o8h, aggo, dego,
                 srcv, dstv, gidxv, rowsv, onesv, z8v, acc, dacc, sem):
    c = lax.axis_index("c")
    s = lax.axis_index("s")
    wid = s * NC + c

    pltpu.sync_copy(zrh, rowsv)
    pltpu.sync_copy(o8h, onesv)
    pltpu.sync_copy(z8h, z8v)

    rb = s * RPT
    for j in range(NJ):
        pltpu.sync_copy(rowsv, acc.at[pl.ds(rb + j * K, K)])
        pltpu.sync_copy(z8v, dacc.at[pl.ds(rb + j * K, K)])
    plsc.subcore_barrier()

    pltpu.sync_copy(srch.at[wid], srcv)
    pltpu.sync_copy(dsth.at[wid], dstv)
    pltpu.sync_copy(eth.at[wid], gidxv)

    def _idx(g, carry):
        for i in range(K // 16):
            sl = pl.ds(i * 16, 16)
            gidxv[g, sl] = gidxv[g, sl] * NPAD + srcv[g, sl]
        return carry

    lax.fori_loop(0, CPT, _idx, 0)

    def _chunk(g, carry):
        pltpu.async_copy(hflat.at[gidxv.at[g]], rowsv, sem).wait()
        pltpu.sync_copy(rowsv, acc.at[dstv.at[g]], add=True)
        pltpu.sync_copy(onesv, dacc.at[dstv.at[g]], add=True)
        return carry

    lax.fori_loop(0, CPT, _chunk, 0)
    plsc.subcore_barrier()

    for j in range(NJ):
        r0 = rb + j * K
        pltpu.sync_copy(acc.at[pl.ds(r0, K)], rowsv)
        pltpu.sync_copy(rowsv, aggo.at[c, pl.ds(r0, K)])
        pltpu.sync_copy(dacc.at[pl.ds(r0, K)], z8v)
        pltpu.sync_copy(z8v, dego.at[c, pl.ds(r0, K)])


_sc_agg = pl.kernel(
    _sc_agg_body,
    out_type=(jax.ShapeDtypeStruct((NC, NPAD, DH), jnp.float32),
              jax.ShapeDtypeStruct((NC, NPAD, 8), jnp.float32)),
    mesh=plsc.VectorSubcoreMesh(core_axis_name="c", subcore_axis_name="s",
                                num_cores=NC, num_subcores=NS),
    scratch_types=[
        pltpu.VMEM((CPT, K), jnp.int32),
        pltpu.VMEM((CPT, K), jnp.int32),
        pltpu.VMEM((CPT, K), jnp.int32),
        pltpu.VMEM((K, DH), jnp.float32),
        pltpu.VMEM((K, 8), jnp.float32),
        pltpu.VMEM((K, 8), jnp.float32),
        pltpu.VMEM_SHARED((NPAD, DH), jnp.float32),
        pltpu.VMEM_SHARED((NPAD, 8), jnp.float32),
        pltpu.SemaphoreType.DMA,
    ],
    compiler_params=pltpu.CompilerParams(use_tc_tiling_on_sc=False),
)


def _mm_body(x_ref, w_ref, oa_ref, ob_ref):
    res = jnp.dot(x_ref[...], w_ref[0], preferred_element_type=jnp.float32)
    oa_ref[0] = res[:, :DH]
    ob_ref[0] = res[:, DH:]


def _mm(xp, w_all):
    return pl.pallas_call(
        _mm_body,
        grid=(NPAD // BN, 9),
        in_specs=[pl.BlockSpec((BN, D), lambda nb, r: (nb, 0)),
                  pl.BlockSpec((1, D, D), lambda nb, r: (r, 0, 0))],
        out_specs=[pl.BlockSpec((1, BN, DH), lambda nb, r: (r, nb, 0)),
                   pl.BlockSpec((1, BN, DH), lambda nb, r: (r, nb, 0))],
        out_shape=[jax.ShapeDtypeStruct((9, NPAD, DH), jnp.float32),
                   jax.ShapeDtypeStruct((9, NPAD, DH), jnp.float32)],
    )(xp, w_all)


def _combine_body(agg_ref, deg_ref, root_ref, b_ref, o_ref, *, act):
    d = deg_ref[0] + deg_ref[1]
    degv = jnp.sum(d, axis=1) * 0.125
    inv = 1.0 / jnp.maximum(degv, 1.0)
    h = (agg_ref[0] + agg_ref[1]) * inv[:, None] + root_ref[...] + b_ref[...]
    o_ref[...] = jnp.maximum(h, 0.0) if act else h


def _combine(agg, deg, root, b2d, act):
    return pl.pallas_call(
        functools.partial(_combine_body, act=act),
        grid=(NPAD // BN,),
        in_specs=[pl.BlockSpec((NC, BN, DH), lambda nb: (0, nb, 0)),
                  pl.BlockSpec((NC, BN, 8), lambda nb: (0, nb, 0)),
                  pl.BlockSpec((BN, DH), lambda nb: (nb, 0)),
                  pl.BlockSpec((1, DH), lambda nb: (0, 0))],
        out_specs=pl.BlockSpec((BN, DH), lambda nb: (nb, 0)),
        out_shape=jax.ShapeDtypeStruct((NPAD, DH), jnp.float32),
    )(agg, deg, root, b2d)


def _layer(xp, w_all, b, src2, dst2, et2, consts, deg_in, act):
    zr, z8, o8 = consts
    ha, hb = _mm(xp, w_all)
    agga, dega = _sc_agg(ha.reshape(9 * NPAD, DH), src2, dst2, et2, zr, z8, o8)
    aggb, _ = _sc_agg(hb.reshape(9 * NPAD, DH), src2, dst2, et2, zr, z8, o8)
    deg = dega if deg_in is None else deg_in
    oa = _combine(agga, deg, ha[8], b[:DH].reshape(1, DH), act)
    ob = _combine(aggb, deg, hb[8], b[DH:].reshape(1, DH), act)
    return jnp.concatenate([oa, ob], axis=1), deg


def kernel(x, edge_index, edge_type, W_rel1, W_root1, b1, W_rel2, W_root2, b2):
    f32 = jnp.float32
    src2 = edge_index[0].astype(jnp.int32).reshape(NT, CPT, K)
    dst2 = edge_index[1].astype(jnp.int32).reshape(NT, CPT, K)
    et2 = edge_type.astype(jnp.int32).reshape(NT, CPT, K)
    xp = jnp.pad(x.astype(f32), ((0, NPAD - N), (0, 0)))
    w_all1 = jnp.concatenate([W_rel1, W_root1[None]], axis=0).astype(f32)
    w_all2 = jnp.concatenate([W_rel2, W_root2[None]], axis=0).astype(f32)
    consts = (jnp.zeros((K, DH), f32), jnp.zeros((K, 8), f32),
              jnp.ones((K, 8), f32))

    h, deg = _layer(xp, w_all1, b1, src2, dst2, et2, consts, None, True)
    out, _ = _layer(h, w_all2, b2, src2, dst2, et2, consts, deg, False)
    return out[:N]


# speedup vs baseline: 16.4885x; 1.4258x over previous
import functools

import jax
import jax.numpy as jnp
from jax import lax
from jax.experimental import pallas as pl
from jax.experimental.pallas import tpu as pltpu
from jax.experimental.pallas import tpu_sc as plsc

N = 10000
E = 320000
D = 128
DH = 64
NPAD = 10240
NC, NS = 2, 16
NT = NC * NS
K = 80
CPT = E // (NT * K)
RPT = NPAD // NS
NJ = RPT // K
BN = 1024


def _make_sc_body(with_deg):
    def body(hflat, srch, dsth, eth, zrh, z8h, o8h, *refs):
        if with_deg:
            (aggo, dego, srcv, dstv, gidxv, rows0, rows1, onesv, z8v, acc,
             dacc, sem0, sem1) = refs
        else:
            aggo, srcv, dstv, gidxv, rows0, rows1, acc, sem0, sem1 = refs
        c = lax.axis_index("c")
        s = lax.axis_index("s")
        wid = s * NC + c

        pltpu.sync_copy(zrh, rows0)
        if with_deg:
            pltpu.sync_copy(o8h, onesv)
            pltpu.sync_copy(z8h, z8v)
        rb = s * RPT
        for j in range(NJ):
            pltpu.sync_copy(rows0, acc.at[pl.ds(rb + j * K, K)])
            if with_deg:
                pltpu.sync_copy(z8v, dacc.at[pl.ds(rb + j * K, K)])
        plsc.subcore_barrier()

        pltpu.sync_copy(srch.at[wid], srcv)
        pltpu.sync_copy(dsth.at[wid], dstv)
        pltpu.sync_copy(eth.at[wid], gidxv)

        def _idx(g, carry):
            for i in range(K // 16):
                sl = pl.ds(i * 16, 16)
                gidxv[g, sl] = gidxv[g, sl] * NPAD + srcv[g, sl]
            return carry

        lax.fori_loop(0, CPT, _idx, 0)

        def _start(g, buf, sem):
            pltpu.async_copy(hflat.at[gidxv.at[g]], buf, sem)

        def _wait(buf, sem):
            pltpu.make_async_copy(hflat.at[pl.ds(0, K)], buf, sem).wait()

        def _scat(g, buf):
            pltpu.sync_copy(buf, acc.at[dstv.at[g]], add=True)
            if with_deg:
                pltpu.sync_copy(onesv, dacc.at[dstv.at[g]], add=True)

        _start(0, rows0, sem0)
        _start(1, rows1, sem1)

        def _pair(j, carry):
            g0 = 2 * j
            _wait(rows0, sem0)
            _scat(g0, rows0)
            _start(g0 + 2, rows0, sem0)
            _wait(rows1, sem1)
            _scat(g0 + 1, rows1)

            @pl.when(g0 + 3 < CPT)
            def _():
                _start(g0 + 3, rows1, sem1)

            return carry

        lax.fori_loop(0, (CPT - 1) // 2, _pair, 0)
        _wait(rows0, sem0)
        _scat(CPT - 1, rows0)
        plsc.subcore_barrier()

        for j in range(NJ):
            r0 = rb + j * K
            pltpu.sync_copy(acc.at[pl.ds(r0, K)], rows0)
            pltpu.sync_copy(rows0, aggo.at[c, pl.ds(r0, K)])
            if with_deg:
                pltpu.sync_copy(dacc.at[pl.ds(r0, K)], z8v)
                pltpu.sync_copy(z8v, dego.at[c, pl.ds(r0, K)])

    return body


def _make_sc_agg(with_deg):
    out_type = [jax.ShapeDtypeStruct((NC, NPAD, DH), jnp.float32)]
    scratch = [
        pltpu.VMEM((CPT, K), jnp.int32),
        pltpu.VMEM((CPT, K), jnp.int32),
        pltpu.VMEM((CPT, K), jnp.int32),
        pltpu.VMEM((K, DH), jnp.float32),
        pltpu.VMEM((K, DH), jnp.float32),
    ]
    if with_deg:
        out_type.append(jax.ShapeDtypeStruct((NC, NPAD, 8), jnp.float32))
        scratch += [pltpu.VMEM((K, 8), jnp.float32),
                    pltpu.VMEM((K, 8), jnp.float32)]
    scratch.append(pltpu.VMEM_SHARED((NPAD, DH), jnp.float32))
    if with_deg:
        scratch.append(pltpu.VMEM_SHARED((NPAD, 8), jnp.float32))
    scratch += [pltpu.SemaphoreType.DMA, pltpu.SemaphoreType.DMA]
    return pl.kernel(
        _make_sc_body(with_deg),
        out_type=tuple(out_type),
        mesh=plsc.VectorSubcoreMesh(core_axis_name="c", subcore_axis_name="s",
                                    num_cores=NC, num_subcores=NS),
        scratch_types=scratch,
        compiler_params=pltpu.CompilerParams(use_tc_tiling_on_sc=False),
    )


_sc_agg_deg = _make_sc_agg(True)
_sc_agg = _make_sc_agg(False)


def _mm_body(x_ref, w_ref, oa_ref, ob_ref):
    res = jnp.dot(x_ref[...], w_ref[0], preferred_element_type=jnp.float32)
    oa_ref[0] = res[:, :DH]
    ob_ref[0] = res[:, DH:]


def _mm(xp, w_all):
    return pl.pallas_call(
        _mm_body,
        grid=(NPAD // BN, 9),
        in_specs=[pl.BlockSpec((BN, D), lambda nb, r: (nb, 0)),
                  pl.BlockSpec((1, D, D), lambda nb, r: (r, 0, 0))],
        out_specs=[pl.BlockSpec((1, BN, DH), lambda nb, r: (r, nb, 0)),
                   pl.BlockSpec((1, BN, DH), lambda nb, r: (r, nb, 0))],
        out_shape=[jax.ShapeDtypeStruct((9, NPAD, DH), jnp.float32),
                   jax.ShapeDtypeStruct((9, NPAD, DH), jnp.float32)],
    )(xp, w_all)


def _combine_body(agg_ref, deg_ref, root_ref, b_ref, o_ref, *, act):
    d = deg_ref[0] + deg_ref[1]
    degv = jnp.sum(d, axis=1) * 0.125
    inv = 1.0 / jnp.maximum(degv, 1.0)
    h = (agg_ref[0] + agg_ref[1]) * inv[:, None] + root_ref[...] + b_ref[...]
    o_ref[...] = jnp.maximum(h, 0.0) if act else h


def _combine(agg, deg, root, b2d, act):
    return pl.pallas_call(
        functools.partial(_combine_body, act=act),
        grid=(NPAD // BN,),
        in_specs=[pl.BlockSpec((NC, BN, DH), lambda nb: (0, nb, 0)),
                  pl.BlockSpec((NC, BN, 8), lambda nb: (0, nb, 0)),
                  pl.BlockSpec((BN, DH), lambda nb: (nb, 0)),
                  pl.BlockSpec((1, DH), lambda nb: (0, 0))],
        out_specs=pl.BlockSpec((BN, DH), lambda nb: (nb, 0)),
        out_shape=jax.ShapeDtypeStruct((NPAD, DH), jnp.float32),
    )(agg, deg, root, b2d)


def _layer(xp, w_all, b, src2, dst2, et2, consts, deg_in, act):
    zr, z8, o8 = consts
    ha, hb = _mm(xp, w_all)
    if deg_in is None:
        agga, deg = _sc_agg_deg(ha.reshape(9 * NPAD, DH), src2, dst2, et2,
                                zr, z8, o8)
    else:
        (agga,) = _sc_agg(ha.reshape(9 * NPAD, DH), src2, dst2, et2,
                          zr, z8, o8)
        deg = deg_in
    (aggb,) = _sc_agg(hb.reshape(9 * NPAD, DH), src2, dst2, et2, zr, z8, o8)
    oa = _combine(agga, deg, ha[8], b[:DH].reshape(1, DH), act)
    ob = _combine(aggb, deg, hb[8], b[DH:].reshape(1, DH), act)
    return jnp.concatenate([oa, ob], axis=1), deg


def kernel(x, edge_index, edge_type, W_rel1, W_root1, b1, W_rel2, W_root2, b2):
    f32 = jnp.float32
    src2 = edge_index[0].astype(jnp.int32).reshape(NT, CPT, K)
    dst2 = edge_index[1].astype(jnp.int32).reshape(NT, CPT, K)
    et2 = edge_type.astype(jnp.int32).reshape(NT, CPT, K)
    xp = jnp.pad(x.astype(f32), ((0, NPAD - N), (0, 0)))
    w_all1 = jnp.concatenate([W_rel1, W_root1[None]], axis=0).astype(f32)
    w_all2 = jnp.concatenate([W_rel2, W_root2[None]], axis=0).astype(f32)
    consts = (jnp.zeros((K, DH), f32), jnp.zeros((K, 8), f32),
              jnp.ones((K, 8), f32))

    h, deg = _layer(xp, w_all1, b1, src2, dst2, et2, consts, None, True)
    out, _ = _layer(h, w_all2, b2, src2, dst2, et2, consts, deg, False)
    return out[:N]


# speedup vs baseline: 19.5056x; 1.1830x over previous
import functools

import jax
import jax.numpy as jnp
from jax import lax
from jax.experimental import pallas as pl
from jax.experimental.pallas import tpu as pltpu
from jax.experimental.pallas import tpu_sc as plsc

N = 10000
E = 320000
D = 128
DH = 64
NPAD = 10240
NC, NS = 2, 16
NT = NC * NS
K = 80
CPT = E // (NT * K)
RPT = NPAD // NS
NJ = RPT // K
BN = 1024


def _make_sc_body(with_deg):
    def body(hflat, srch, dsth, eth, zrh, z8h, o8h, *refs):
        if with_deg:
            (aggo, dego, srcv, dstv, gidxv, rows0, rows1, rows2, rows3,
             onesv, z8v, acc, dacc, sem0, sem1, sem2, sem3) = refs
        else:
            (aggo, srcv, dstv, gidxv, rows0, rows1, rows2, rows3, acc,
             sem0, sem1, sem2, sem3) = refs
        bufs = ((rows0, sem0), (rows1, sem1), (rows2, sem2), (rows3, sem3))
        nbuf = len(bufs)
        c = lax.axis_index("c")
        s = lax.axis_index("s")
        wid = s * NC + c

        pltpu.sync_copy(zrh, rows0)
        if with_deg:
            pltpu.sync_copy(o8h, onesv)
            pltpu.sync_copy(z8h, z8v)
        rb = s * RPT
        for j in range(NJ):
            pltpu.sync_copy(rows0, acc.at[pl.ds(rb + j * K, K)])
            if with_deg:
                pltpu.sync_copy(z8v, dacc.at[pl.ds(rb + j * K, K)])
        plsc.subcore_barrier()

        pltpu.sync_copy(srch.at[wid], srcv)
        pltpu.sync_copy(dsth.at[wid], dstv)
        pltpu.sync_copy(eth.at[wid], gidxv)

        def _idx(g, carry):
            for i in range(K // 16):
                sl = pl.ds(i * 16, 16)
                gidxv[g, sl] = gidxv[g, sl] * NPAD + srcv[g, sl]
            return carry

        lax.fori_loop(0, CPT, _idx, 0)

        def _start(g, buf, sem):
            pltpu.async_copy(hflat.at[gidxv.at[g]], buf, sem)

        def _wait(buf, sem):
            pltpu.make_async_copy(hflat.at[pl.ds(0, K)], buf, sem).wait()

        def _scat(g, buf):
            pltpu.sync_copy(buf, acc.at[dstv.at[g]], add=True)
            if with_deg:
                pltpu.sync_copy(onesv, dacc.at[dstv.at[g]], add=True)

        for b, (buf, sem) in enumerate(bufs):
            _start(b, buf, sem)

        def _group(j, carry):
            for b, (buf, sem) in enumerate(bufs):
                g = nbuf * j + b
                _wait(buf, sem)
                _scat(g, buf)

                @pl.when(g + nbuf < CPT)
                def _():
                    _start(g + nbuf, buf, sem)

            return carry

        lax.fori_loop(0, CPT // nbuf, _group, 0)
        for b, (buf, sem) in enumerate(bufs):
            g = nbuf * (CPT // nbuf) + b
            if g < CPT:
                _wait(buf, sem)
                _scat(g, buf)
        plsc.subcore_barrier()

        for j in range(NJ):
            r0 = rb + j * K
            pltpu.sync_copy(acc.at[pl.ds(r0, K)], rows0)
            pltpu.sync_copy(rows0, aggo.at[c, pl.ds(r0, K)])
            if with_deg:
                pltpu.sync_copy(dacc.at[pl.ds(r0, K)], z8v)
                pltpu.sync_copy(z8v, dego.at[c, pl.ds(r0, K)])

    return body


def _make_sc_agg(with_deg):
    out_type = [jax.ShapeDtypeStruct((NC, NPAD, DH), jnp.float32)]
    scratch = [
        pltpu.VMEM((CPT, K), jnp.int32),
        pltpu.VMEM((CPT, K), jnp.int32),
        pltpu.VMEM((CPT, K), jnp.int32),
        pltpu.VMEM((K, DH), jnp.float32),
        pltpu.VMEM((K, DH), jnp.float32),
        pltpu.VMEM((K, DH), jnp.float32),
        pltpu.VMEM((K, DH), jnp.float32),
    ]
    if with_deg:
        out_type.append(jax.ShapeDtypeStruct((NC, NPAD, 8), jnp.float32))
        scratch += [pltpu.VMEM((K, 8), jnp.float32),
                    pltpu.VMEM((K, 8), jnp.float32)]
    scratch.append(pltpu.VMEM_SHARED((NPAD, DH), jnp.float32))
    if with_deg:
        scratch.append(pltpu.VMEM_SHARED((NPAD, 8), jnp.float32))
    scratch += [pltpu.SemaphoreType.DMA, pltpu.SemaphoreType.DMA,
                pltpu.SemaphoreType.DMA, pltpu.SemaphoreType.DMA]
    return pl.kernel(
        _make_sc_body(with_deg),
        out_type=tuple(out_type),
        mesh=plsc.VectorSubcoreMesh(core_axis_name="c", subcore_axis_name="s",
                                    num_cores=NC, num_subcores=NS),
        scratch_types=scratch,
        compiler_params=pltpu.CompilerParams(use_tc_tiling_on_sc=False),
    )


_sc_agg_deg = _make_sc_agg(True)
_sc_agg = _make_sc_agg(False)


def _mm_body(x_ref, w_ref, oa_ref, ob_ref):
    res = jnp.dot(x_ref[...], w_ref[0], preferred_element_type=jnp.float32)
    oa_ref[0] = res[:, :DH]
    ob_ref[0] = res[:, DH:]


def _mm(xp, w_all):
    return pl.pallas_call(
        _mm_body,
        grid=(NPAD // BN, 9),
        in_specs=[pl.BlockSpec((BN, D), lambda nb, r: (nb, 0)),
                  pl.BlockSpec((1, D, D), lambda nb, r: (r, 0, 0))],
        out_specs=[pl.BlockSpec((1, BN, DH), lambda nb, r: (r, nb, 0)),
                   pl.BlockSpec((1, BN, DH), lambda nb, r: (r, nb, 0))],
        out_shape=[jax.ShapeDtypeStruct((9, NPAD, DH), jnp.float32),
                   jax.ShapeDtypeStruct((9, NPAD, DH), jnp.float32)],
    )(xp, w_all)


def _combine_body(agg_ref, deg_ref, root_ref, b_ref, o_ref, *, act):
    d = deg_ref[0] + deg_ref[1]
    degv = jnp.sum(d, axis=1) * 0.125
    inv = 1.0 / jnp.maximum(degv, 1.0)
    h = (agg_ref[0] + agg_ref[1]) * inv[:, None] + root_ref[...] + b_ref[...]
    o_ref[...] = jnp.maximum(h, 0.0) if act else h


def _combine(agg, deg, root, b2d, act):
    return pl.pallas_call(
        functools.partial(_combine_body, act=act),
        grid=(NPAD // BN,),
        in_specs=[pl.BlockSpec((NC, BN, DH), lambda nb: (0, nb, 0)),
                  pl.BlockSpec((NC, BN, 8), lambda nb: (0, nb, 0)),
                  pl.BlockSpec((BN, DH), lambda nb: (nb, 0)),
                  pl.BlockSpec((1, DH), lambda nb: (0, 0))],
        out_specs=pl.BlockSpec((BN, DH), lambda nb: (nb, 0)),
        out_shape=jax.ShapeDtypeStruct((NPAD, DH), jnp.float32),
    )(agg, deg, root, b2d)


def _layer(xp, w_all, b, src2, dst2, et2, consts, deg_in, act):
    zr, z8, o8 = consts
    ha, hb = _mm(xp, w_all)
    if deg_in is None:
        agga, deg = _sc_agg_deg(ha.reshape(9 * NPAD, DH), src2, dst2, et2,
                                zr, z8, o8)
    else:
        (agga,) = _sc_agg(ha.reshape(9 * NPAD, DH), src2, dst2, et2,
                          zr, z8, o8)
        deg = deg_in
    (aggb,) = _sc_agg(hb.reshape(9 * NPAD, DH), src2, dst2, et2, zr, z8, o8)
    oa = _combine(agga, deg, ha[8], b[:DH].reshape(1, DH), act)
    ob = _combine(aggb, deg, hb[8], b[DH:].reshape(1, DH), act)
    return jnp.concatenate([oa, ob], axis=1), deg


def kernel(x, edge_index, edge_type, W_rel1, W_root1, b1, W_rel2, W_root2, b2):
    f32 = jnp.float32
    src2 = edge_index[0].astype(jnp.int32).reshape(NT, CPT, K)
    dst2 = edge_index[1].astype(jnp.int32).reshape(NT, CPT, K)
    et2 = edge_type.astype(jnp.int32).reshape(NT, CPT, K)
    xp = jnp.pad(x.astype(f32), ((0, NPAD - N), (0, 0)))
    w_all1 = jnp.concatenate([W_rel1, W_root1[None]], axis=0).astype(f32)
    w_all2 = jnp.concatenate([W_rel2, W_root2[None]], axis=0).astype(f32)
    consts = (jnp.zeros((K, DH), f32), jnp.zeros((K, 8), f32),
              jnp.ones((K, 8), f32))

    h, deg = _layer(xp, w_all1, b1, src2, dst2, et2, consts, None, True)
    out, _ = _layer(h, w_all2, b2, src2, dst2, et2, consts, deg, False)
    return out[:N]
